# baseline (speedup 1.0000x reference)
import functools

import jax
import jax.numpy as jnp
from jax.experimental import pallas as pl
from jax.experimental.pallas import tpu as pltpu


def _pcat_kernel(feat_ref, w_ref, b_ref, pcat_ref, *, d):
    p = jnp.dot(feat_ref[...], w_ref[...].T,
                preferred_element_type=jnp.float32) + b_ref[...]
    pcat_ref[:, :d] = p
    pcat_ref[:, d:] = p * p


def _bilinear_kernel(adj_ref, pcat_ref, x_ref, *, d):
    sq = jnp.dot(adj_ref[...], pcat_ref[...],
                 preferred_element_type=jnp.float32)
    s = sq[:, :d]
    q = sq[:, d:]
    x_ref[...] = 0.5 * (s * s - q)


def _diag_kernel(diag_ref, x_ref, out_ref):
    out_ref[...] = jnp.dot(diag_ref[...], x_ref[...],
                           preferred_element_type=jnp.float32)


def kernel(feat, adj_loop, diag_mat, W, b):
    n, _ = feat.shape
    d = W.shape[0]
    bm = 400
    g = n // bm

    pcat = pl.pallas_call(
        functools.partial(_pcat_kernel, d=d),
        out_shape=jax.ShapeDtypeStruct((n, 2 * d), jnp.float32),
    )(feat, W, b.reshape(1, d))

    x = pl.pallas_call(
        functools.partial(_bilinear_kernel, d=d),
        grid=(g,),
        in_specs=[
            pl.BlockSpec((bm, n), lambda i: (i, 0)),
            pl.BlockSpec((n, 2 * d), lambda i: (0, 0)),
        ],
        out_specs=pl.BlockSpec((bm, d), lambda i: (i, 0)),
        out_shape=jax.ShapeDtypeStruct((n, d), jnp.float32),
        compiler_params=pltpu.CompilerParams(
            dimension_semantics=("parallel",)),
    )(adj_loop, pcat)

    out = pl.pallas_call(
        _diag_kernel,
        grid=(g,),
        in_specs=[
            pl.BlockSpec((bm, n), lambda i: (i, 0)),
            pl.BlockSpec((n, d), lambda i: (0, 0)),
        ],
        out_specs=pl.BlockSpec((bm, d), lambda i: (i, 0)),
        out_shape=jax.ShapeDtypeStruct((n, d), jnp.float32),
        compiler_params=pltpu.CompilerParams(
            dimension_semantics=("parallel",)),
    )(diag_mat, x)

    return out


# speedup vs baseline: 1.0596x; 1.0596x over previous
import functools

import jax
import jax.numpy as jnp
from jax.experimental import pallas as pl
from jax.experimental.pallas import tpu as pltpu

_DEPTH = 2
_NSPLIT = 2


def _chunks(bm, nsplit):
    base = (bm // nsplit) // 8 * 8
    sizes = [base] * (nsplit - 1)
    sizes.append(bm - base * (nsplit - 1))
    offs, o = [], 0
    for s in sizes:
        offs.append(o)
        o += s
    return list(zip(offs, sizes))


def _fused_kernel(feat_ref, w_ref, b_ref, adj_hbm, diag_hbm, out_ref,
                  pcat_ref, x_ref, *rest, g, bm, d):
    bufs, sem = rest[:-1], rest[-1]
    i = pl.program_id(0)

    chunks = _chunks(bm, _NSPLIT)

    def issue(j, k):
        @pl.when(j < g)
        def _():
            for p, (o, sz) in enumerate(chunks):
                pltpu.make_async_copy(
                    adj_hbm.at[pl.ds(j * bm + o, sz), :],
                    bufs[k].at[pl.ds(o, sz), :], sem.at[k, p]).start()

        @pl.when(jnp.logical_and(j >= g, j < 2 * g))
        def _():
            for p, (o, sz) in enumerate(chunks):
                pltpu.make_async_copy(
                    diag_hbm.at[pl.ds((j - g) * bm + o, sz), :],
                    bufs[k].at[pl.ds(o, sz), :], sem.at[k, p]).start()

    @pl.when(i == 0)
    def _prologue():
        for k in range(_DEPTH):
            issue(jnp.int32(k), k)
        p = jnp.dot(feat_ref[...], w_ref[...].T,
                    preferred_element_type=jnp.float32) + b_ref[...]
        pcat_ref[:, :d] = p
        pcat_ref[:, d:] = p * p

    slot = jax.lax.rem(i, _DEPTH)

    def step_body(k):
        for p, (o, sz) in enumerate(chunks):
            pltpu.make_async_copy(adj_hbm.at[pl.ds(0, sz), :],
                                  bufs[k].at[pl.ds(o, sz), :],
                                  sem.at[k, p]).wait()

        @pl.when(i < g)
        def _phase_adj():
            sq = jnp.dot(bufs[k][...], pcat_ref[...],
                         preferred_element_type=jnp.float32)
            s = sq[:, :d]
            q = sq[:, d:]
            x_ref[pl.ds(i * bm, bm), :] = 0.5 * (s * s - q)

        @pl.when(i >= g)
        def _phase_diag():
            out_ref[...] = jnp.dot(bufs[k][...], x_ref[...],
                                   preferred_element_type=jnp.float32)

        issue(i + _DEPTH, k)

    for k in range(_DEPTH):
        pl.when(slot == k)(functools.partial(step_body, k))


def kernel(feat, adj_loop, diag_mat, W, b):
    n, _ = feat.shape
    d = W.shape[0]
    bm = 400 if n % 400 == 0 else n
    g = n // bm

    return pl.pallas_call(
        functools.partial(_fused_kernel, g=g, bm=bm, d=d),
        grid=(2 * g,),
        in_specs=[
            pl.BlockSpec((n, feat.shape[1]), lambda i: (0, 0)),
            pl.BlockSpec((d, W.shape[1]), lambda i: (0, 0)),
            pl.BlockSpec((1, d), lambda i: (0, 0)),
            pl.BlockSpec(memory_space=pltpu.MemorySpace.HBM),
            pl.BlockSpec(memory_space=pltpu.MemorySpace.HBM),
        ],
        out_specs=pl.BlockSpec((bm, d), lambda i: (jnp.maximum(i - g, 0), 0)),
        out_shape=jax.ShapeDtypeStruct((n, d), jnp.float32),
        scratch_shapes=[
            pltpu.VMEM((n, 2 * d), jnp.float32),
            pltpu.VMEM((n, d), jnp.float32),
            *[pltpu.VMEM((bm, n), jnp.float32) for _ in range(_DEPTH)],
            pltpu.SemaphoreType.DMA((_DEPTH, _NSPLIT)),
        ],
    )(feat, W, b.reshape(1, d), adj_loop, diag_mat)
